# baseline (speedup 1.0000x reference)
import jax
import jax.numpy as jnp
from jax.experimental import pallas as pl

_B = 4096
_D_IN = 768
_H1 = 512
_H2 = 256
_LATENT = 128
_NPROTO = 250
_K = 15
_PCOLS = 256
_BLK = 512
_BIG = 3e38


def _fused_body(x_ref, W1_ref, b1_ref, g1_ref, bt1_ref,
                W2_ref, b2_ref, g2_ref, bt2_ref,
                W3_ref, b3_ref, pT_ref, out_ref):
    f32 = jnp.float32
    h = jnp.dot(x_ref[...], W1_ref[...], preferred_element_type=f32) + b1_ref[...]
    m = jnp.mean(h, axis=-1, keepdims=True)
    v = jnp.mean((h - m) * (h - m), axis=-1, keepdims=True)
    h = (h - m) / jnp.sqrt(v + 1e-5) * g1_ref[...] + bt1_ref[...]
    h = jnp.maximum(h, 0.0)

    h = jnp.dot(h, W2_ref[...], preferred_element_type=f32) + b2_ref[...]
    m = jnp.mean(h, axis=-1, keepdims=True)
    v = jnp.mean((h - m) * (h - m), axis=-1, keepdims=True)
    h = (h - m) / jnp.sqrt(v + 1e-5) * g2_ref[...] + bt2_ref[...]
    h = jnp.maximum(h, 0.0)

    z = jnp.maximum(
        jnp.dot(h, W3_ref[...], preferred_element_type=f32) + b3_ref[...], 0.0)

    P = pT_ref[...]
    ptz = jax.lax.dot_general(P, z, (((1,), (1,)), ((), ())),
                              preferred_element_type=f32)
    q = z * z
    ones8 = jnp.ones((8, _LATENT), f32)
    z2r = jax.lax.dot_general(ones8, q, (((1,), (1,)), ((), ())),
                              preferred_element_type=f32)[0:1]
    p2c = jnp.sum(P * P, axis=1, keepdims=True)
    d2 = z2r + p2c - 2.0 * ptz

    row = jax.lax.broadcasted_iota(jnp.int32, (_PCOLS, _BLK), 0)
    d = jnp.sqrt(jnp.maximum(d2, 1e-12))

    ibits = jax.lax.bitcast_convert_type(d, jnp.int32)
    cur = jax.lax.bitcast_convert_type((ibits & ~jnp.int32(0xFF)) | row, f32)

    acc = jnp.zeros((1, _BLK), f32)
    dmin = None
    for k in range(_K):
        mval = jnp.min(cur, axis=0, keepdims=True)
        if dmin is None:
            dmin = mval
        acc = acc + mval
        if k < _K - 1:
            cur = jnp.where(cur == mval, _BIG, cur)

    score = 0.1 * (acc * (1.0 / _K)) + 0.9 * dmin
    out_ref[...] = (1.0 - jnp.exp(-0.3 * score))[None]


@jax.jit
def kernel(x, W1, b1, g1, bt1, W2, b2, g2, bt2, W3, b3, prototypes):
    pT = jnp.full((_PCOLS, _LATENT), -1e15, jnp.float32).at[:_NPROTO].set(prototypes)
    grid = (_B // _BLK,)
    row_blk = lambda i: (i, 0)
    full = lambda i: (0, 0)
    out = pl.pallas_call(
        _fused_body,
        grid=grid,
        in_specs=[
            pl.BlockSpec((_BLK, _D_IN), row_blk),
            pl.BlockSpec((_D_IN, _H1), full),
            pl.BlockSpec((1, _H1), full),
            pl.BlockSpec((1, _H1), full),
            pl.BlockSpec((1, _H1), full),
            pl.BlockSpec((_H1, _H2), full),
            pl.BlockSpec((1, _H2), full),
            pl.BlockSpec((1, _H2), full),
            pl.BlockSpec((1, _H2), full),
            pl.BlockSpec((_H2, _LATENT), full),
            pl.BlockSpec((1, _LATENT), full),
            pl.BlockSpec((_PCOLS, _LATENT), full),
        ],
        out_specs=pl.BlockSpec((1, 1, _BLK), lambda i: (i, 0, 0)),
        out_shape=jax.ShapeDtypeStruct((_B // _BLK, 1, _BLK), jnp.float32),
    )(x, W1, b1.reshape(1, -1), g1.reshape(1, -1), bt1.reshape(1, -1),
      W2, b2.reshape(1, -1), g2.reshape(1, -1), bt2.reshape(1, -1),
      W3, b3.reshape(1, -1), pT)
    return out.reshape(_B)


# speedup vs baseline: 1.1584x; 1.1584x over previous
import jax
import jax.numpy as jnp
from jax.experimental import pallas as pl

_B = 4096
_D_IN = 768
_H1 = 512
_H2 = 256
_LATENT = 128
_NPROTO = 250
_K = 15
_PCOLS = 256
_BLK = 1024
_BIG = 3e38


def _fused_body(x_ref, W1_ref, b1_ref, g1_ref, bt1_ref,
                W2_ref, b2_ref, g2_ref, bt2_ref,
                W3_ref, b3_ref, pT_ref, out_ref):
    f32 = jnp.float32
    h = jnp.dot(x_ref[...], W1_ref[...], preferred_element_type=f32) + b1_ref[...]
    m = jnp.mean(h, axis=-1, keepdims=True)
    v = jnp.mean((h - m) * (h - m), axis=-1, keepdims=True)
    h = (h - m) / jnp.sqrt(v + 1e-5) * g1_ref[...] + bt1_ref[...]
    h = jnp.maximum(h, 0.0)

    h = jnp.dot(h, W2_ref[...], preferred_element_type=f32) + b2_ref[...]
    m = jnp.mean(h, axis=-1, keepdims=True)
    v = jnp.mean((h - m) * (h - m), axis=-1, keepdims=True)
    h = (h - m) / jnp.sqrt(v + 1e-5) * g2_ref[...] + bt2_ref[...]
    h = jnp.maximum(h, 0.0)

    z = jnp.maximum(
        jnp.dot(h, W3_ref[...], preferred_element_type=f32) + b3_ref[...], 0.0)

    P = pT_ref[...]
    ptz = jax.lax.dot_general(P, z, (((1,), (1,)), ((), ())),
                              preferred_element_type=f32)
    q = z * z
    ones8 = jnp.ones((8, _LATENT), f32)
    z2r = jax.lax.dot_general(ones8, q, (((1,), (1,)), ((), ())),
                              preferred_element_type=f32)[0:1]
    p2c = jnp.sum(P * P, axis=1, keepdims=True)
    d2 = z2r + p2c - 2.0 * ptz

    row = jax.lax.broadcasted_iota(jnp.int32, (_PCOLS, _BLK), 0)
    d = jnp.sqrt(jnp.maximum(d2, 1e-12))

    ibits = jax.lax.bitcast_convert_type(d, jnp.int32)
    cur = jax.lax.bitcast_convert_type((ibits & ~jnp.int32(0xFF)) | row, f32)

    lo = jnp.minimum(cur[:128], cur[128:])
    hi = jnp.maximum(cur[:128], cur[128:])
    acc = jnp.zeros((1, _BLK), f32)
    dmin = None
    for k in range(_K):
        mval = jnp.min(lo, axis=0, keepdims=True)
        if dmin is None:
            dmin = mval
        acc = acc + mval
        if k < _K - 1:
            sel = lo == mval
            lo = jnp.where(sel, hi, lo)
            hi = jnp.where(sel, _BIG, hi)

    score = 0.1 * (acc * (1.0 / _K)) + 0.9 * dmin
    out_ref[...] = (1.0 - jnp.exp(-0.3 * score))[None]


@jax.jit
def kernel(x, W1, b1, g1, bt1, W2, b2, g2, bt2, W3, b3, prototypes):
    pT = jnp.full((_PCOLS, _LATENT), -1e15, jnp.float32).at[:_NPROTO].set(prototypes)
    grid = (_B // _BLK,)
    row_blk = lambda i: (i, 0)
    full = lambda i: (0, 0)
    out = pl.pallas_call(
        _fused_body,
        grid=grid,
        in_specs=[
            pl.BlockSpec((_BLK, _D_IN), row_blk),
            pl.BlockSpec((_D_IN, _H1), full),
            pl.BlockSpec((1, _H1), full),
            pl.BlockSpec((1, _H1), full),
            pl.BlockSpec((1, _H1), full),
            pl.BlockSpec((_H1, _H2), full),
            pl.BlockSpec((1, _H2), full),
            pl.BlockSpec((1, _H2), full),
            pl.BlockSpec((1, _H2), full),
            pl.BlockSpec((_H2, _LATENT), full),
            pl.BlockSpec((1, _LATENT), full),
            pl.BlockSpec((_PCOLS, _LATENT), full),
        ],
        out_specs=pl.BlockSpec((1, 1, _BLK), lambda i: (i, 0, 0)),
        out_shape=jax.ShapeDtypeStruct((_B // _BLK, 1, _BLK), jnp.float32),
    )(x, W1, b1.reshape(1, -1), g1.reshape(1, -1), bt1.reshape(1, -1),
      W2, b2.reshape(1, -1), g2.reshape(1, -1), bt2.reshape(1, -1),
      W3, b3.reshape(1, -1), pT)
    return out.reshape(_B)
